# Initial kernel scaffold; baseline (speedup 1.0000x reference)
import jax
import jax.numpy as jnp
from jax.experimental import pallas as pl


def kernel(msas, features, seq_lens, focuses, term_lens, src_key_mask, chain_idx, coords, aa_embed, W_feat, W_res, rbf_centers, W_e, W_msg, W_eupd):
    raise NotImplementedError("write your pallas kernel here")


# speedup vs baseline: 8.1767x; 8.1767x over previous
import functools

import jax
import jax.numpy as jnp
from jax.experimental import pallas as pl
from jax.experimental.pallas import tpu as pltpu

_K = 16

def _dotx(a, b):
    return jnp.dot(a, b, precision=jax.lax.Precision.HIGHEST)


def _dot(a, b):
    return jnp.dot(a, b, precision=None)


def _mpnn_kernel(msas_ref, feat_ref, focr_ref, focc_ref, ca_ref,
                 aa_ref, wf_ref, wr_ref, cent_ref, we_ref, wmsg_ref, weupd_ref,
                 agg_ref, edges_ref, acc_ref, cnt_ref,
                 *, NTt, T, H, S, V, nsteps):
    s = pl.program_id(1)
    Nt = NTt * T
    f32 = jnp.float32

    @pl.when(s == 0)
    def _init():
        acc_ref[...] = jnp.zeros_like(acc_ref)
        cnt_ref[...] = jnp.zeros_like(cnt_ref)

    msas = msas_ref[0, 0]
    feats = feat_ref[0, 0]
    focr = focr_ref[0, 0]
    focc = focc_ref[0, 0]
    caS = ca_ref[0]

    oh_aa = (msas == jax.lax.broadcasted_iota(jnp.int32, (Nt, V), 1)).astype(f32)
    emb = _dot(oh_aa, aa_ref[...]) + _dot(feats, wf_ref[...])
    conv = emb + jax.nn.relu(_dot(emb, wr_ref[...]))

    oh_g = (focc == jax.lax.broadcasted_iota(jnp.int32, (Nt, S), 1)).astype(f32)
    ca = _dotx(oh_g, caS)

    wmsg = wmsg_ref[...]
    weupd = weupd_ref[...]
    we = we_ref[...]
    WrM = _dot(we, wmsg[2 * H:3 * H])
    WrE = _dot(we, weupd[2 * H:3 * H])
    hiM = _dot(conv, wmsg[0:H])
    hiE = _dot(conv, weupd[0:H])
    btM = _dot(conv, wmsg[H:2 * H])
    btE = _dot(conv, weupd[H:2 * H])

    oh_term = (jax.lax.broadcasted_iota(jnp.int32, (Nt, NTt), 0) // T
               == jax.lax.broadcasted_iota(jnp.int32, (Nt, NTt), 1)).astype(f32)
    selbase = (jax.lax.broadcasted_iota(jnp.int32, (NTt, Nt), 1)
               - jax.lax.broadcasted_iota(jnp.int32, (NTt, Nt), 0) * T)

    dcols = []
    for j in range(T):
        selj = (selbase == j).astype(f32)
        cbj = _dotx(oh_term, _dotx(selj, ca))
        diff = ca - cbj
        dcols.append(jnp.sqrt(jnp.sum(diff * diff, axis=1, keepdims=True) + 1e-8))
    d = jnp.concatenate(dcols, axis=1)

    lane_j = jax.lax.broadcasted_iota(jnp.int32, (Nt, T), 1)
    cent = cent_ref[...]

    node_sum = jnp.zeros((Nt, H), f32)
    for j in range(T):
        dj = d[:, j:j + 1]
        rank = jnp.sum((d < dj).astype(f32)
                       + ((d == dj) & (lane_j < j)).astype(f32),
                       axis=1, keepdims=True)
        selv = (rank < _K).astype(f32)
        selj = (selbase == j).astype(f32)
        hjM = _dot(oh_term, _dot(selj, btM))
        hjE = _dot(oh_term, _dot(selj, btE))
        rbf = jnp.exp(-jnp.square(dj - cent) * 0.125)
        msg = jax.nn.relu(hiM + hjM + _dot(rbf, WrM))
        edge = jax.nn.relu(hiE + hjE + _dot(rbf, WrE))
        node_sum = node_sum + selv * msg
        edges_ref[0, :, j, :] = selv * edge

    node_emb = conv + node_sum * (1.0 / _K)

    oh_s = (focr == jax.lax.broadcasted_iota(jnp.int32, (S, Nt), 0)).astype(f32)
    acc_ref[...] += _dot(oh_s, node_emb)
    cnt_ref[...] += jnp.sum(oh_s, axis=1, keepdims=True)

    @pl.when(s == nsteps - 1)
    def _fin():
        c = cnt_ref[...]
        agg_ref[0] = acc_ref[...] / jnp.where(c == 0.0, 1.0, c)


def kernel(msas, features, seq_lens, focuses, term_lens, src_key_mask,
           chain_idx, coords, aa_embed, W_feat, W_res, rbf_centers,
           W_e, W_msg, W_eupd):
    B, N = msas.shape
    H = aa_embed.shape[1]
    V = aa_embed.shape[0]
    NT = term_lens.shape[1]
    T = N // NT
    S = coords.shape[1]
    F = features.shape[-1]

    NTt = 20
    nsteps = NT // NTt
    Nt = NTt * T

    msas_r = msas.reshape(B, nsteps, Nt, 1).astype(jnp.int32)
    feats_r = features.reshape(B, nsteps, Nt, F)
    focr = focuses.reshape(B, nsteps, 1, Nt).astype(jnp.int32)
    focc = focuses.reshape(B, nsteps, Nt, 1).astype(jnp.int32)
    ca = coords[:, :, 1, :]
    cent = rbf_centers.reshape(1, H)

    grid = (B, nsteps)
    kfn = functools.partial(_mpnn_kernel, NTt=NTt, T=T, H=H, S=S, V=V,
                            nsteps=nsteps)
    agg, edges = pl.pallas_call(
        kfn,
        grid=grid,
        in_specs=[
            pl.BlockSpec((1, 1, Nt, 1), lambda b, s: (b, s, 0, 0)),
            pl.BlockSpec((1, 1, Nt, F), lambda b, s: (b, s, 0, 0)),
            pl.BlockSpec((1, 1, 1, Nt), lambda b, s: (b, s, 0, 0)),
            pl.BlockSpec((1, 1, Nt, 1), lambda b, s: (b, s, 0, 0)),
            pl.BlockSpec((1, S, 3), lambda b, s: (b, 0, 0)),
            pl.BlockSpec((V, H), lambda b, s: (0, 0)),
            pl.BlockSpec((F, H), lambda b, s: (0, 0)),
            pl.BlockSpec((H, H), lambda b, s: (0, 0)),
            pl.BlockSpec((1, H), lambda b, s: (0, 0)),
            pl.BlockSpec((H, H), lambda b, s: (0, 0)),
            pl.BlockSpec((3 * H, H), lambda b, s: (0, 0)),
            pl.BlockSpec((3 * H, H), lambda b, s: (0, 0)),
        ],
        out_specs=[
            pl.BlockSpec((1, S, H), lambda b, s: (b, 0, 0)),
            pl.BlockSpec((1, Nt, T, H), lambda b, s: (b, s, 0, 0)),
        ],
        out_shape=[
            jax.ShapeDtypeStruct((B, S, H), jnp.float32),
            jax.ShapeDtypeStruct((B, N, T, H), jnp.float32),
        ],
        scratch_shapes=[
            pltpu.VMEM((S, H), jnp.float32),
            pltpu.VMEM((S, 1), jnp.float32),
        ],
        compiler_params=pltpu.CompilerParams(
            dimension_semantics=("arbitrary", "arbitrary")),
    )(msas_r, feats_r, focr, focc, ca, aa_embed, W_feat, W_res, cent,
      W_e, W_msg, W_eupd)

    return agg, edges.reshape(B, NT, T, T, H)
